# Initial kernel scaffold; baseline (speedup 1.0000x reference)
import jax
import jax.numpy as jnp
from jax.experimental import pallas as pl


def kernel(h, edge_index, W1, b1, W2, b2):
    raise NotImplementedError("write your pallas kernel here")


# speedup vs baseline: 9.2497x; 9.2497x over previous
import functools

import jax
import jax.numpy as jnp
from jax import lax
from jax.experimental import pallas as pl
from jax.experimental.pallas import tpu as pltpu
from jax.experimental.pallas import tpu_sc as plsc

N_NODES = 10000
N_PAD = 10240
N_EDGES = 320000
D = 128

NC = 2
NS = 16
CHUNK = 80
EDGE_ROWS = N_EDGES // CHUNK
ROWS_PER_TILE_DEG = EDGE_ROWS // NS
EPT_AGG = N_EDGES // (NC * NS)
ROWS_PER_TILE_AGG = EPT_AGG // CHUNK
SLAB = N_PAD // NS

_mesh = plsc.VectorSubcoreMesh(core_axis_name="c", subcore_axis_name="s")


@functools.partial(
    pl.kernel,
    out_type=jax.ShapeDtypeStruct((NC, N_PAD, D), jnp.float32),
    mesh=_mesh,
    scratch_types=[
        pltpu.VMEM((ROWS_PER_TILE_DEG, CHUNK), jnp.int32),
        pltpu.VMEM((CHUNK, D), jnp.float32),
        pltpu.VMEM_SHARED((N_PAD, D), jnp.float32),
    ],
)
def _deg_kernel(idx2d_hbm, ones_hbm, zeros_hbm, out_hbm, idx_v, ones_v, acc_sh):
    cid = lax.axis_index("c")
    sid = lax.axis_index("s")
    pltpu.sync_copy(idx2d_hbm.at[cid, sid], idx_v)
    pltpu.sync_copy(ones_hbm, ones_v)
    pltpu.sync_copy(zeros_hbm, acc_sh.at[pl.ds(sid * SLAB, SLAB)])
    plsc.subcore_barrier()

    def body(j, carry):
        pltpu.sync_copy(ones_v, acc_sh.at[idx_v.at[j]], add=True)
        return carry

    lax.fori_loop(0, ROWS_PER_TILE_DEG, body, 0)
    plsc.subcore_barrier()
    pltpu.sync_copy(
        acc_sh.at[pl.ds(sid * SLAB, SLAB)],
        out_hbm.at[cid, pl.ds(sid * SLAB, SLAB)],
    )


@functools.partial(
    pl.kernel,
    out_type=jax.ShapeDtypeStruct((NC, N_PAD, D), jnp.float32),
    mesh=_mesh,
    scratch_types=[
        pltpu.VMEM((EPT_AGG,), jnp.int32),
        pltpu.VMEM((ROWS_PER_TILE_AGG, CHUNK), jnp.int32),
        pltpu.VMEM((CHUNK, D), jnp.float32),
        pltpu.VMEM((CHUNK, D), jnp.float32),
        pltpu.VMEM_SHARED((N_PAD, D), jnp.float32),
        pltpu.SemaphoreType.DMA,
        pltpu.SemaphoreType.DMA,
    ],
)
def _agg_kernel(t_hbm, src_hbm, dst3d_hbm, zeros_hbm, out_hbm,
                src_v, dst_v, buf_a, buf_b, acc_sh, sem_a, sem_b):
    cid = lax.axis_index("c")
    sid = lax.axis_index("s")
    wid = cid * NS + sid
    pltpu.sync_copy(src_hbm.at[pl.ds(wid * EPT_AGG, EPT_AGG)], src_v)
    pltpu.sync_copy(dst3d_hbm.at[wid], dst_v)
    pltpu.sync_copy(zeros_hbm, acc_sh.at[pl.ds(sid * SLAB, SLAB)])
    plsc.subcore_barrier()

    first = pltpu.async_copy(t_hbm.at[src_v.at[pl.ds(0, CHUNK)]], buf_a, sem_a)

    def body(j, carry):
        @pl.when(j % 2 == 0)
        def _even():
            @pl.when(j + 1 < ROWS_PER_TILE_AGG)
            def _pf():
                pltpu.async_copy(
                    t_hbm.at[src_v.at[pl.ds((j + 1) * CHUNK, CHUNK)]], buf_b, sem_b)
            pltpu.make_async_copy(t_hbm.at[pl.ds(0, CHUNK)], buf_a, sem_a).wait()
            pltpu.sync_copy(buf_a, acc_sh.at[dst_v.at[j]], add=True)

        @pl.when(j % 2 == 1)
        def _odd():
            @pl.when(j + 1 < ROWS_PER_TILE_AGG)
            def _pf():
                pltpu.async_copy(
                    t_hbm.at[src_v.at[pl.ds((j + 1) * CHUNK, CHUNK)]], buf_a, sem_a)
            pltpu.make_async_copy(t_hbm.at[pl.ds(0, CHUNK)], buf_b, sem_b).wait()
            pltpu.sync_copy(buf_b, acc_sh.at[dst_v.at[j]], add=True)

        return carry

    lax.fori_loop(0, ROWS_PER_TILE_AGG, body, 0)
    plsc.subcore_barrier()
    pltpu.sync_copy(
        acc_sh.at[pl.ds(sid * SLAB, SLAB)],
        out_hbm.at[cid, pl.ds(sid * SLAB, SLAB)],
    )


NB = 1280


def _rsqrt_clip(deg_col):
    return lax.rsqrt(jnp.maximum(deg_col, 1.0))


def _tc1_body(h_ref, deg_ref, w1_ref, o_ref):
    y = lax.dot_general(h_ref[...], w1_ref[...], (((0,), (0,)), ((), ())),
                        preferred_element_type=jnp.float32)
    ns = _rsqrt_clip(deg_ref[0, :, 0:1])
    o_ref[...] = y * ns


def _tc2_body(p_ref, deg_ref, b1_ref, o_ref):
    s = p_ref[0] + p_ref[1]
    ns = _rsqrt_clip(deg_ref[0, :, 0:1])
    nd = _rsqrt_clip(deg_ref[1, :, 0:1])
    o_ref[...] = ns * jnp.maximum(s * nd + b1_ref[...], 0.0)


def _tc3_body(p_ref, deg_ref, w2_ref, b2_ref, o_ref):
    nd = _rsqrt_clip(deg_ref[1, :, 0:1])
    s = (p_ref[0] + p_ref[1]) * nd
    o_ref[...] = lax.dot_general(w2_ref[...], s, (((0,), (1,)), ((), ())),
                                 preferred_element_type=jnp.float32) + b2_ref[...]


_deg_spec = pl.BlockSpec((NC, NB, D), lambda j: (0, j, 0))
_part_spec = pl.BlockSpec((NC, NB, D), lambda j: (0, j, 0))

_tc1 = pl.pallas_call(
    _tc1_body,
    grid=(N_PAD // NB,),
    in_specs=[
        pl.BlockSpec((D, NB), lambda j: (0, j)),
        _deg_spec,
        pl.BlockSpec((D, D), lambda j: (0, 0)),
    ],
    out_specs=pl.BlockSpec((NB, D), lambda j: (j, 0)),
    out_shape=jax.ShapeDtypeStruct((N_PAD, D), jnp.float32),
)

_tc2 = pl.pallas_call(
    _tc2_body,
    grid=(N_PAD // NB,),
    in_specs=[
        _part_spec,
        _deg_spec,
        pl.BlockSpec((1, D), lambda j: (0, 0)),
    ],
    out_specs=pl.BlockSpec((NB, D), lambda j: (j, 0)),
    out_shape=jax.ShapeDtypeStruct((N_PAD, D), jnp.float32),
)

_tc3 = pl.pallas_call(
    _tc3_body,
    grid=(N_PAD // NB,),
    in_specs=[
        _part_spec,
        _deg_spec,
        pl.BlockSpec((D, D), lambda j: (0, 0)),
        pl.BlockSpec((D, 1), lambda j: (0, 0)),
    ],
    out_specs=pl.BlockSpec((D, NB), lambda j: (0, j)),
    out_shape=jax.ShapeDtypeStruct((D, N_PAD), jnp.float32),
)


@jax.jit
def kernel(h, edge_index, W1, b1, W2, b2):
    src = edge_index[0]
    idx4d = edge_index.reshape(2, NS, ROWS_PER_TILE_DEG, CHUNK)
    dst3d = edge_index[1].reshape(NC * NS, ROWS_PER_TILE_AGG, CHUNK)
    hp = jnp.pad(h, ((0, 0), (0, N_PAD - N_NODES)))
    ones128 = jnp.ones((CHUNK, D), jnp.float32)
    zeros128 = jnp.zeros((SLAB, D), jnp.float32)

    degs = _deg_kernel(idx4d, ones128, zeros128)
    t1 = _tc1(hp, degs, W1)
    a1 = _agg_kernel(t1, src, dst3d, zeros128)
    t2 = _tc2(a1, degs, b1.reshape(1, D))
    a2 = _agg_kernel(t2, src, dst3d, zeros128)
    out = _tc3(a2, degs, W2, b2.reshape(D, 1))
    return out[:, :N_NODES]


# speedup vs baseline: 9.3287x; 1.0085x over previous
import functools

import jax
import jax.numpy as jnp
from jax import lax
from jax.experimental import pallas as pl
from jax.experimental.pallas import tpu as pltpu
from jax.experimental.pallas import tpu_sc as plsc

N_NODES = 10000
N_PAD = 10240
N_EDGES = 320000
D = 128

NC = 2
NS = 16
CHUNK = 80
EDGE_ROWS = N_EDGES // CHUNK
ROWS_PER_TILE_DEG = EDGE_ROWS // NS
EPT_AGG = N_EDGES // (NC * NS)
ROWS_PER_TILE_AGG = EPT_AGG // CHUNK
SLAB = N_PAD // NS

_mesh = plsc.VectorSubcoreMesh(core_axis_name="c", subcore_axis_name="s")


@functools.partial(
    pl.kernel,
    out_type=jax.ShapeDtypeStruct((NC, N_PAD, D), jnp.float32),
    mesh=_mesh,
    scratch_types=[
        pltpu.VMEM((ROWS_PER_TILE_DEG, CHUNK), jnp.int32),
        pltpu.VMEM((CHUNK, D), jnp.float32),
        pltpu.VMEM_SHARED((N_PAD, D), jnp.float32),
        pltpu.SemaphoreType.DMA,
    ],
)
def _deg_kernel(idx2d_hbm, ones_hbm, zeros_hbm, out_hbm, idx_v, ones_v, acc_sh, sem_s):
    cid = lax.axis_index("c")
    sid = lax.axis_index("s")
    pltpu.sync_copy(idx2d_hbm.at[cid, sid], idx_v)
    pltpu.sync_copy(ones_hbm, ones_v)
    pltpu.sync_copy(zeros_hbm, acc_sh.at[pl.ds(sid * SLAB, SLAB)])
    plsc.subcore_barrier()

    DEPTH = 8

    def body(j, carry):
        pltpu.async_copy(ones_v, acc_sh.at[idx_v.at[j]], sem_s, add=True)

        @pl.when(j >= DEPTH)
        def _drain():
            pltpu.make_async_copy(zeros_hbm.at[pl.ds(0, CHUNK)], ones_v, sem_s).wait()

        return carry

    lax.fori_loop(0, ROWS_PER_TILE_DEG, body, 0)

    def drain(j, carry):
        pltpu.make_async_copy(zeros_hbm.at[pl.ds(0, CHUNK)], ones_v, sem_s).wait()
        return carry

    lax.fori_loop(0, DEPTH, drain, 0)
    plsc.subcore_barrier()
    pltpu.sync_copy(
        acc_sh.at[pl.ds(sid * SLAB, SLAB)],
        out_hbm.at[cid, pl.ds(sid * SLAB, SLAB)],
    )


@functools.partial(
    pl.kernel,
    out_type=jax.ShapeDtypeStruct((NC, N_PAD, D), jnp.float32),
    mesh=_mesh,
    scratch_types=[
        pltpu.VMEM((EPT_AGG,), jnp.int32),
        pltpu.VMEM((ROWS_PER_TILE_AGG, CHUNK), jnp.int32),
        pltpu.VMEM((CHUNK, D), jnp.float32),
        pltpu.VMEM((CHUNK, D), jnp.float32),
        pltpu.VMEM_SHARED((N_PAD, D), jnp.float32),
        pltpu.SemaphoreType.DMA,
        pltpu.SemaphoreType.DMA,
    ],
)
def _agg_kernel(t_hbm, src_hbm, dst3d_hbm, zeros_hbm, out_hbm,
                src_v, dst_v, buf0, buf1, acc_sh, sg0, sg1):
    bufs = [buf0, buf1]
    sem_g = [sg0, sg1]
    cid = lax.axis_index("c")
    sid = lax.axis_index("s")
    wid = cid * NS + sid
    pltpu.sync_copy(src_hbm.at[pl.ds(wid * EPT_AGG, EPT_AGG)], src_v)
    pltpu.sync_copy(dst3d_hbm.at[wid], dst_v)
    pltpu.sync_copy(zeros_hbm, acc_sh.at[pl.ds(sid * SLAB, SLAB)])
    plsc.subcore_barrier()

    NCH = ROWS_PER_TILE_AGG

    def gather(j, b):
        pltpu.async_copy(t_hbm.at[src_v.at[pl.ds(j * CHUNK, CHUNK)]],
                         bufs[b], sem_g[b])

    def wait_bytes(b, sem):
        pltpu.make_async_copy(zeros_hbm.at[pl.ds(0, CHUNK)], bufs[b], sem[b]).wait()

    gather(0, 0)

    def body(j, carry):
        for b in range(2):
            @pl.when(j % 2 == b)
            def _do(b=b):
                @pl.when(j + 1 < NCH)
                def _pf():
                    gather(j + 1, 1 - b)
                wait_bytes(b, sem_g)
                pltpu.sync_copy(bufs[b], acc_sh.at[dst_v.at[j]], add=True)
        return carry

    lax.fori_loop(0, NCH, body, 0)
    plsc.subcore_barrier()
    pltpu.sync_copy(
        acc_sh.at[pl.ds(sid * SLAB, SLAB)],
        out_hbm.at[cid, pl.ds(sid * SLAB, SLAB)],
    )


NB = 1280


def _rsqrt_clip(deg_col):
    return lax.rsqrt(jnp.maximum(deg_col, 1.0))


def _tc1a_body(h_ref, w1_ref, o_ref):
    o_ref[...] = lax.dot_general(h_ref[...], w1_ref[...], (((0,), (0,)), ((), ())),
                                 preferred_element_type=jnp.float32)


def _tc1b_body(y_ref, deg_ref, o_ref):
    ns = _rsqrt_clip(deg_ref[0, :, 0:1])
    o_ref[...] = y_ref[...] * ns


def _tc2_body(p_ref, deg_ref, b1_ref, o_ref):
    s = p_ref[0] + p_ref[1]
    ns = _rsqrt_clip(deg_ref[0, :, 0:1])
    nd = _rsqrt_clip(deg_ref[1, :, 0:1])
    o_ref[...] = ns * jnp.maximum(s * nd + b1_ref[...], 0.0)


def _tc3_body(p_ref, deg_ref, w2_ref, b2_ref, o_ref):
    nd = _rsqrt_clip(deg_ref[1, :, 0:1])
    s = (p_ref[0] + p_ref[1]) * nd
    o_ref[...] = lax.dot_general(w2_ref[...], s, (((0,), (1,)), ((), ())),
                                 preferred_element_type=jnp.float32) + b2_ref[...]


_deg_spec = pl.BlockSpec((NC, NB, D), lambda j: (0, j, 0))
_part_spec = pl.BlockSpec((NC, NB, D), lambda j: (0, j, 0))

_tc1a = pl.pallas_call(
    _tc1a_body,
    grid=(N_PAD // NB,),
    in_specs=[
        pl.BlockSpec((D, NB), lambda j: (0, j)),
        pl.BlockSpec((D, D), lambda j: (0, 0)),
    ],
    out_specs=pl.BlockSpec((NB, D), lambda j: (j, 0)),
    out_shape=jax.ShapeDtypeStruct((N_PAD, D), jnp.float32),
)

_tc1b = pl.pallas_call(
    _tc1b_body,
    grid=(N_PAD // NB,),
    in_specs=[
        pl.BlockSpec((NB, D), lambda j: (j, 0)),
        _deg_spec,
    ],
    out_specs=pl.BlockSpec((NB, D), lambda j: (j, 0)),
    out_shape=jax.ShapeDtypeStruct((N_PAD, D), jnp.float32),
)

_tc2 = pl.pallas_call(
    _tc2_body,
    grid=(N_PAD // NB,),
    in_specs=[
        _part_spec,
        _deg_spec,
        pl.BlockSpec((1, D), lambda j: (0, 0)),
    ],
    out_specs=pl.BlockSpec((NB, D), lambda j: (j, 0)),
    out_shape=jax.ShapeDtypeStruct((N_PAD, D), jnp.float32),
)

_tc3 = pl.pallas_call(
    _tc3_body,
    grid=(N_PAD // NB,),
    in_specs=[
        _part_spec,
        _deg_spec,
        pl.BlockSpec((D, D), lambda j: (0, 0)),
        pl.BlockSpec((D, 1), lambda j: (0, 0)),
    ],
    out_specs=pl.BlockSpec((D, NB), lambda j: (0, j)),
    out_shape=jax.ShapeDtypeStruct((D, N_PAD), jnp.float32),
)


@jax.jit
def kernel(h, edge_index, W1, b1, W2, b2):
    src = edge_index[0]
    idx4d = edge_index.reshape(2, NS, ROWS_PER_TILE_DEG, CHUNK)
    dst3d = edge_index[1].reshape(NC * NS, ROWS_PER_TILE_AGG, CHUNK)
    hp = jnp.pad(h, ((0, 0), (0, N_PAD - N_NODES)))
    ones128 = jnp.ones((CHUNK, D), jnp.float32)
    zeros128 = jnp.zeros((SLAB, D), jnp.float32)

    degs = _deg_kernel(idx4d, ones128, zeros128)
    y1 = _tc1a(hp, W1)
    t1 = _tc1b(y1, degs)
    a1 = _agg_kernel(t1, src, dst3d, zeros128)
    t2 = _tc2(a1, degs, b1.reshape(1, D))
    a2 = _agg_kernel(t2, src, dst3d, zeros128)
    out = _tc3(a2, degs, W2, b2.reshape(D, 1))
    return out[:, :N_NODES]


# speedup vs baseline: 9.3538x; 1.0027x over previous
import functools

import jax
import jax.numpy as jnp
from jax import lax
from jax.experimental import pallas as pl
from jax.experimental.pallas import tpu as pltpu
from jax.experimental.pallas import tpu_sc as plsc

N_NODES = 10000
N_PAD = 10240
N_EDGES = 320000
D = 128

NC = 2
NS = 16
CHUNK = 80
EDGE_ROWS = N_EDGES // CHUNK
ROWS_PER_TILE_DEG = EDGE_ROWS // NS
EPT_AGG = N_EDGES // (NC * NS)
ROWS_PER_TILE_AGG = EPT_AGG // CHUNK
SLAB = N_PAD // NS

_mesh = plsc.VectorSubcoreMesh(core_axis_name="c", subcore_axis_name="s")


@functools.partial(
    pl.kernel,
    out_type=jax.ShapeDtypeStruct((NC, N_PAD, D), jnp.float32),
    mesh=_mesh,
    scratch_types=[
        pltpu.VMEM((ROWS_PER_TILE_DEG, CHUNK), jnp.int32),
        pltpu.VMEM((CHUNK, D), jnp.float32),
        pltpu.VMEM_SHARED((N_PAD, D), jnp.float32),
        pltpu.SemaphoreType.DMA,
    ],
)
def _deg_kernel(idx2d_hbm, ones_hbm, zeros_hbm, out_hbm, idx_v, ones_v, acc_sh, sem_s):
    cid = lax.axis_index("c")
    sid = lax.axis_index("s")
    pltpu.sync_copy(idx2d_hbm.at[cid, sid], idx_v)
    pltpu.sync_copy(ones_hbm, ones_v)
    pltpu.sync_copy(zeros_hbm, acc_sh.at[pl.ds(sid * SLAB, SLAB)])
    plsc.subcore_barrier()

    DEPTH = 8

    def body(j, carry):
        pltpu.async_copy(ones_v, acc_sh.at[idx_v.at[j]], sem_s, add=True)

        @pl.when(j >= DEPTH)
        def _drain():
            pltpu.make_async_copy(zeros_hbm.at[pl.ds(0, CHUNK)], ones_v, sem_s).wait()

        return carry

    lax.fori_loop(0, ROWS_PER_TILE_DEG, body, 0)

    def drain(j, carry):
        pltpu.make_async_copy(zeros_hbm.at[pl.ds(0, CHUNK)], ones_v, sem_s).wait()
        return carry

    lax.fori_loop(0, DEPTH, drain, 0)
    plsc.subcore_barrier()
    pltpu.sync_copy(
        acc_sh.at[pl.ds(sid * SLAB, SLAB)],
        out_hbm.at[cid, pl.ds(sid * SLAB, SLAB)],
    )


@functools.partial(
    pl.kernel,
    out_type=jax.ShapeDtypeStruct((NC, N_PAD, D), jnp.float32),
    mesh=_mesh,
    scratch_types=[
        pltpu.VMEM((EPT_AGG,), jnp.int32),
        pltpu.VMEM((ROWS_PER_TILE_AGG, CHUNK), jnp.int32),
        pltpu.VMEM((CHUNK, D), jnp.float32),
        pltpu.VMEM((CHUNK, D), jnp.float32),
        pltpu.VMEM_SHARED((N_PAD, D), jnp.float32),
        pltpu.SemaphoreType.DMA,
        pltpu.SemaphoreType.DMA,
    ],
)
def _agg_kernel(t_hbm, src_hbm, dst3d_hbm, zeros_hbm, out_hbm,
                src_v, dst_v, buf0, buf1, acc_sh, sg0, sg1):
    sem_g = [sg0, sg1]
    bufs2 = [buf0, buf1]
    cid = lax.axis_index("c")
    sid = lax.axis_index("s")
    wid = cid * NS + sid
    pltpu.sync_copy(src_hbm.at[pl.ds(wid * EPT_AGG, EPT_AGG)], src_v)
    pltpu.sync_copy(dst3d_hbm.at[wid], dst_v)
    pltpu.sync_copy(zeros_hbm.at[pl.ds(0, CHUNK)], buf0)
    for k in range(SLAB // CHUNK):
        pltpu.sync_copy(buf0, acc_sh.at[pl.ds(sid * SLAB + k * CHUNK, CHUNK)])
    plsc.subcore_barrier()

    NCH = ROWS_PER_TILE_AGG

    def gather(j, b):
        pltpu.async_copy(t_hbm.at[src_v.at[pl.ds(j * CHUNK, CHUNK)]],
                         bufs2[b], sem_g[b])

    def wait_bytes(b, sem):
        pltpu.make_async_copy(zeros_hbm.at[pl.ds(0, CHUNK)], bufs2[b], sem[b]).wait()

    gather(0, 0)

    def body(j, carry):
        for b in range(2):
            @pl.when(j % 2 == b)
            def _do(b=b):
                @pl.when(j + 1 < NCH)
                def _pf():
                    gather(j + 1, 1 - b)
                wait_bytes(b, sem_g)
                pltpu.sync_copy(bufs2[b], acc_sh.at[dst_v.at[j]], add=True)
        return carry

    lax.fori_loop(0, NCH, body, 0)
    plsc.subcore_barrier()
    pltpu.sync_copy(
        acc_sh.at[pl.ds(sid * SLAB, SLAB)],
        out_hbm.at[cid, pl.ds(sid * SLAB, SLAB)],
    )


NB = 1280


def _rsqrt_clip(deg_col):
    return lax.rsqrt(jnp.maximum(deg_col, 1.0))


def _tc1a_body(h_ref, w1_ref, o_ref):
    o_ref[...] = lax.dot_general(h_ref[...], w1_ref[...], (((0,), (0,)), ((), ())),
                                 preferred_element_type=jnp.float32)


def _tc1b_body(y_ref, deg_ref, o_ref):
    ns = _rsqrt_clip(deg_ref[0, :, 0:1])
    o_ref[...] = y_ref[...] * ns


def _tc2_body(p_ref, deg_ref, b1_ref, o_ref):
    s = p_ref[0] + p_ref[1]
    ns = _rsqrt_clip(deg_ref[0, :, 0:1])
    nd = _rsqrt_clip(deg_ref[1, :, 0:1])
    o_ref[...] = ns * jnp.maximum(s * nd + b1_ref[...], 0.0)


def _tc3_body(p_ref, deg_ref, w2_ref, b2_ref, o_ref):
    nd = _rsqrt_clip(deg_ref[1, :, 0:1])
    s = (p_ref[0] + p_ref[1]) * nd
    o_ref[...] = lax.dot_general(w2_ref[...], s, (((0,), (1,)), ((), ())),
                                 preferred_element_type=jnp.float32) + b2_ref[...]


_deg_spec = pl.BlockSpec((NC, NB, D), lambda j: (0, j, 0))
_part_spec = pl.BlockSpec((NC, NB, D), lambda j: (0, j, 0))

_tc1a = pl.pallas_call(
    _tc1a_body,
    grid=(N_PAD // NB,),
    in_specs=[
        pl.BlockSpec((D, NB), lambda j: (0, j)),
        pl.BlockSpec((D, D), lambda j: (0, 0)),
    ],
    out_specs=pl.BlockSpec((NB, D), lambda j: (j, 0)),
    out_shape=jax.ShapeDtypeStruct((N_PAD, D), jnp.float32),
)

_tc1b = pl.pallas_call(
    _tc1b_body,
    grid=(N_PAD // NB,),
    in_specs=[
        pl.BlockSpec((NB, D), lambda j: (j, 0)),
        _deg_spec,
    ],
    out_specs=pl.BlockSpec((NB, D), lambda j: (j, 0)),
    out_shape=jax.ShapeDtypeStruct((N_PAD, D), jnp.float32),
)

_tc2 = pl.pallas_call(
    _tc2_body,
    grid=(N_PAD // NB,),
    in_specs=[
        _part_spec,
        _deg_spec,
        pl.BlockSpec((1, D), lambda j: (0, 0)),
    ],
    out_specs=pl.BlockSpec((NB, D), lambda j: (j, 0)),
    out_shape=jax.ShapeDtypeStruct((N_PAD, D), jnp.float32),
)

_tc3 = pl.pallas_call(
    _tc3_body,
    grid=(N_PAD // NB,),
    in_specs=[
        _part_spec,
        _deg_spec,
        pl.BlockSpec((D, D), lambda j: (0, 0)),
        pl.BlockSpec((D, 1), lambda j: (0, 0)),
    ],
    out_specs=pl.BlockSpec((D, NB), lambda j: (0, j)),
    out_shape=jax.ShapeDtypeStruct((D, N_PAD), jnp.float32),
)


@jax.jit
def kernel(h, edge_index, W1, b1, W2, b2):
    src = edge_index[0]
    idx4d = edge_index.reshape(2, NS, ROWS_PER_TILE_DEG, CHUNK)
    dst3d = edge_index[1].reshape(NC * NS, ROWS_PER_TILE_AGG, CHUNK)
    hp = jnp.pad(h, ((0, 0), (0, N_PAD - N_NODES)))
    ones128 = jnp.ones((CHUNK, D), jnp.float32)
    zeros128 = jnp.zeros((SLAB, D), jnp.float32)

    degs = _deg_kernel(idx4d, ones128, zeros128)
    y1 = _tc1a(hp, W1)
    t1 = _tc1b(y1, degs)
    a1 = _agg_kernel(t1, src, dst3d, zeros128)
    t2 = _tc2(a1, degs, b1.reshape(1, D))
    a2 = _agg_kernel(t2, src, dst3d, zeros128)
    out = _tc3(a2, degs, W2, b2.reshape(D, 1))
    return out[:, :N_NODES]
